# baseline (speedup 1.0000x reference)
import functools

import jax
import jax.numpy as jnp
from jax import lax
from jax.experimental import pallas as pl
from jax.experimental.pallas import tpu as pltpu
from jax.experimental.pallas import tpu_sc as plsc

_D = 64
_V = 1000000
_NC = 2
_NS = 16
_NW = _NC * _NS
_BB = 128
_FULL = (_V // (2 * _BB)) // _NW * _NW * 2
_PER_W = _FULL // _NW
_FULL2 = (_FULL + 4) * _BB


def _relayout_table(table_t, tail_pad):
  mesh = plsc.VectorSubcoreMesh(core_axis_name="c", subcore_axis_name="s")
  iota16 = lambda: lax.iota(jnp.int32, 16)
  full16 = lambda x: jnp.zeros((16,), jnp.int32) + x

  @functools.partial(
      pl.kernel,
      out_type=jax.ShapeDtypeStruct((_V // 2, _BB), jnp.float32),
      mesh=mesh,
      compiler_params=pltpu.CompilerParams(
          needs_layout_passes=False, use_tc_tiling_on_sc=True,
          disable_bounds_checks=True),
      scratch_types=[
          [pltpu.VMEM((_D, _BB), jnp.float32) for _ in range(4)],
          [pltpu.VMEM((_D, _BB), jnp.float32) for _ in range(2)],
          [pltpu.SemaphoreType.DMA for _ in range(4)],
          [pltpu.SemaphoreType.DMA for _ in range(2)],
      ],
  )
  def run(tt_hbm, tp_hbm, tr_hbm, ibuf, obuf, isems, osems):
    wid = lax.axis_index("s") * _NC + lax.axis_index("c")
    base = wid * _PER_W * _BB
    rowv = [iota16() + 16 * g for g in range(4)]

    def in_start(v0, sl):
      off = pl.multiple_of(v0, _BB)
      pltpu.make_async_copy(
          tt_hbm.at[:, pl.ds(off, _BB)], ibuf[sl], isems[sl]).start()

    def in_wait(v0, sl):
      off = pl.multiple_of(v0, _BB)
      pltpu.make_async_copy(
          tt_hbm.at[:, pl.ds(off, _BB)], ibuf[sl], isems[sl]).wait()

    def out_start(v0, sl):
      off = pl.multiple_of(v0 // 2, _D)
      pltpu.make_async_copy(
          obuf[sl], tr_hbm.at[pl.ds(off, _D)], osems[sl]).start()

    def out_wait(v0, sl):
      off = pl.multiple_of(v0 // 2, _D)
      pltpu.make_async_copy(
          obuf[sl], tr_hbm.at[pl.ds(off, _D)], osems[sl]).wait()

    def transpose1(isl, osl):
      def p_body(p, c):
        c0 = full16(2 * p)
        c1 = c0 + 1
        for g in range(8):
          v = plsc.load_gather(ibuf[isl].at[...],
                               [rowv[g % 4], c1 if g >= 4 else c0])
          obuf[osl][p, pl.ds(16 * g, 16)] = v
        return c

      lax.fori_loop(0, _D, p_body, 0, unroll=8)

    in_start(base, 0)
    in_start(base + _BB, 1)
    in_start(base + 2 * _BB, 2)

    def body4(t, carry):
      for u in range(4):
        k = 4 * t + u
        v0 = base + _BB * k
        in_wait(v0, u)

        @pl.when(k + 3 < _PER_W)
        def _nxt(_v0=v0, _sl=(u + 3) % 4):
          in_start(_v0 + 3 * _BB, _sl)

        @pl.when(k >= 2)
        def _drain(_v0=v0, _sl=u % 2):
          out_wait(_v0 - 2 * _BB, _sl)

        transpose1(u, u % 2)

        @pl.when(v0 == 0)
        def _z(_sl=u % 2):
          for g in range(4):
            obuf[_sl][0, pl.ds(16 * g, 16)] = jnp.zeros((16,), jnp.float32)

        out_start(v0, u % 2)
      return carry

    lax.fori_loop(0, _PER_W // 4, body4, 0)
    out_wait(base + _BB * (_PER_W - 2), 0)
    out_wait(base + _BB * (_PER_W - 1), 1)

    @pl.when(wid == 0)
    def _tail():
      for i in range(4):
        v0 = (_FULL + i) * _BB
        sl = i % 2
        in_start(v0, sl)
        in_wait(v0, sl)
        transpose1(sl, sl)
        out_start(v0, sl)
        out_wait(v0, sl)
      pltpu.sync_copy(tp_hbm, ibuf[0])

      def p_body(p, c):
        for g in range(8):
          v = plsc.load_gather(ibuf[0].at[...],
                               [full16(2 * p + (1 if g >= 4 else 0)),
                                rowv[g % 4]])
          obuf[0][p, pl.ds(16 * g, 16)] = v
        return c

      lax.fori_loop(0, (_V - _FULL2) // 2, p_body, 0)
      pltpu.sync_copy(obuf[0].at[pl.ds(0, (_V - _FULL2) // 2)],
                      tr_hbm.at[pl.ds(_FULL2 // 2, (_V - _FULL2) // 2)])

  return run(table_t, tail_pad)


def _embed_lookup(text, table_r, b, s):
  mesh = plsc.VectorSubcoreMesh(core_axis_name="c", subcore_axis_name="s")
  iota16 = lambda: lax.iota(jnp.int32, 16)
  full16 = lambda x: jnp.zeros((16,), jnp.int32) + x
  n_round = 5
  sc = s // n_round
  assert sc % 8 == 0

  @functools.partial(
      pl.kernel,
      out_type=jax.ShapeDtypeStruct((s, _D // 8, _NW, 8, _BB), jnp.float32),
      mesh=mesh,
      compiler_params=pltpu.CompilerParams(
          needs_layout_passes=False, use_tc_tiling_on_sc=False,
          disable_bounds_checks=True),
      scratch_types=[
          pltpu.VMEM((_BB, sc), jnp.int32),
          pltpu.VMEM((s, _BB), jnp.int32),
          pltpu.VMEM((4, _BB), jnp.int32),
          [pltpu.VMEM((_BB, _BB), jnp.float32) for _ in range(4)],
          [pltpu.VMEM((8, 8, _BB), jnp.float32) for _ in range(2)],
          [pltpu.SemaphoreType.DMA for _ in range(4)],
          [pltpu.SemaphoreType.DMA for _ in range(2)],
      ],
  )
  def run(text_hbm, tr_hbm, out_hbm, slab_v, idxT_v, idx2_v, rbuf, tbuf,
          gsems, osems):
    wid = lax.axis_index("s") * _NC + lax.axis_index("c")
    rowv = [iota16() + 16 * g for g in range(8)]

    for r in range(n_round):
      roff = pl.multiple_of(_BB * wid, _BB)
      pltpu.sync_copy(
          text_hbm.at[pl.ds(roff, _BB), pl.ds(sc * r, sc)], slab_v)
      for g in range(8):

        def s_body(q, c, _g=g, _r=r):
          v = plsc.load_gather(slab_v.at[...], [rowv[_g], full16(q)])
          idxT_v[sc * _r + q, pl.ds(16 * _g, 16)] = v
          return c

        lax.fori_loop(0, sc, s_body, 0, unroll=8)

    def fire(q, r):
      for g in range(8):
        idx2_v[r, pl.ds(16 * g, 16)] = (
            idxT_v[q, pl.ds(16 * g, 16)] >> 1)
      pltpu.make_async_copy(
          tr_hbm.at[idx2_v.at[r]], rbuf[r], gsems[r]).start()

    def gwait(q, r):
      pltpu.make_async_copy(
          tr_hbm.at[idx2_v.at[r]], rbuf[r], gsems[r]).wait()

    def out_start(q, ot):
      pltpu.make_async_copy(
          tbuf[ot], out_hbm.at[q, :, wid], osems[ot]).start()

    def out_wait(q, ot):
      pltpu.make_async_copy(
          tbuf[ot], out_hbm.at[q, :, wid], osems[ot]).wait()

    def transpose(q, r, ot):
      colp = [(idxT_v[q, pl.ds(16 * g, 16)] & 1) * _D for g in range(8)]
      for jo in range(8):

        def jr_body(jr, c, _jo=jo):
          j = 8 * _jo + jr
          for g in range(8):
            v = plsc.load_gather(rbuf[r].at[...], [rowv[g], colp[g] + j])
            tbuf[ot][_jo, jr, pl.ds(16 * g, 16)] = v
          return c

        lax.fori_loop(0, 8, jr_body, 0, unroll=4)

    fire(0, 0)
    fire(1, 1)

    def body4(t, carry):
      for bslot in range(4):
        q = 4 * t + bslot
        ot = bslot % 2
        gwait(q, bslot)

        @pl.when(q + 2 < s)
        def _next(_q=q, _b=bslot):
          fire(_q + 2, (_b + 2) % 4)

        @pl.when(q >= 2)
        def _drain(_q=q, _ot=ot):
          out_wait(_q - 2, _ot)

        transpose(q, bslot, ot)
        out_start(q, ot)
      return carry

    lax.fori_loop(0, s // 4, body4, 0)
    out_wait(s - 2, 0)
    out_wait(s - 1, 1)

  return run(text, table_r)


def kernel(text, text_lengths, embedding_weight):
  del text_lengths
  b, s = text.shape
  assert b == _NW * _BB and s % 4 == 0
  tail_pad = jnp.pad(embedding_weight[_FULL2:], ((0, 0), (0, _BB - _D)))
  table_r = _relayout_table(embedding_weight.T, tail_pad)
  out5d = _embed_lookup(text.astype(jnp.int32), table_r, b, s)
  return out5d.transpose((2, 4, 0, 1, 3)).reshape(b, s, _D)


# speedup vs baseline: 2.2919x; 2.2919x over previous
import functools

import jax
import jax.numpy as jnp
from jax import lax
from jax.experimental import pallas as pl
from jax.experimental.pallas import tpu as pltpu
from jax.experimental.pallas import tpu_sc as plsc

_D = 64
_PAD = 0
_NC = 2
_NS = 16
_NW = _NC * _NS
_CHUNK = 256
_NBUF = 4
_LOOK = 2


def _embed_lookup(idx3, table, n_chunks):
  mesh = plsc.VectorSubcoreMesh(core_axis_name="c", subcore_axis_name="s")

  @functools.partial(
      pl.kernel,
      out_type=jax.ShapeDtypeStruct((_NW, n_chunks, _CHUNK, _D), jnp.float32),
      mesh=mesh,
      compiler_params=pltpu.CompilerParams(
          needs_layout_passes=False, use_tc_tiling_on_sc=False),
      scratch_types=[
          pltpu.VMEM((n_chunks, _CHUNK), jnp.int32),
          [pltpu.VMEM((_CHUNK, _D), jnp.float32) for _ in range(_NBUF)],
          pltpu.VMEM((16,), jnp.int32),
          [pltpu.SemaphoreType.DMA for _ in range(_NBUF)],
          [pltpu.SemaphoreType.DMA for _ in range(_NBUF)],
      ],
  )
  def run(idx_hbm, table_hbm, out_hbm, idx_v, rows, flag_v, gsems, osems):
    wid = lax.axis_index("s") * _NC + lax.axis_index("c")
    pltpu.sync_copy(idx_hbm.at[wid], idx_v)

    def fire(j, s):
      pltpu.make_async_copy(table_hbm.at[idx_v.at[j]], rows[s], gsems[s]).start()

    def out_start(j, s):
      pltpu.make_async_copy(rows[s], out_hbm.at[wid, j], osems[s]).start()

    def out_wait(j, s):
      pltpu.make_async_copy(rows[s], out_hbm.at[wid, j], osems[s]).wait()

    def handle(j, s):
      pltpu.make_async_copy(table_hbm.at[idx_v.at[j]], rows[s], gsems[s]).wait()
      idx_row = idx_v.at[j]
      msk_acc = idx_row[pl.ds(0, 16)] == _PAD
      for g in range(1, _CHUNK // 16):
        msk_acc = msk_acc | (idx_row[pl.ds(16 * g, 16)] == _PAD)
      flag_v[...] = jnp.zeros((16,), jnp.int32)
      plsc.store_scatter(flag_v.at[...], [jnp.zeros((16,), jnp.int32)],
                         jnp.ones((16,), jnp.int32), mask=msk_acc)
      nz = flag_v[...][0]

      @pl.when(nz != 0)
      def _fixup():
        zero16 = jnp.zeros((16,), jnp.float32)
        for g in range(_CHUNK // 16):
          v = idx_row[pl.ds(16 * g, 16)]
          msk = v == _PAD
          rowv = 16 * g + lax.iota(jnp.int32, 16)

          def cbody(c, carry, _rowv=rowv, _msk=msk):
            colv = jnp.zeros((16,), jnp.int32) + c
            plsc.store_scatter(rows[s].at[...], [_rowv, colv], zero16,
                               mask=_msk)
            return carry

          lax.fori_loop(0, _D, cbody, 0)

      out_start(j, s)

    for j in range(_LOOK):
      fire(j, j % _NBUF)

    def body4(t, carry):
      for b in range(_NBUF):
        j = _NBUF * t + b
        handle(j, b)
        g = j + _LOOK
        s2 = (b + _LOOK) % _NBUF

        @pl.when(g < n_chunks)
        def _next(_g=g, _s2=s2):
          @pl.when(_g >= _NBUF)
          def _drain():
            out_wait(_g - _NBUF, _s2)

          fire(_g, _s2)

      return carry

    lax.fori_loop(0, n_chunks // _NBUF, body4, 0)

    for b in range(_NBUF):
      out_wait(n_chunks - _NBUF + b, b)

  return run(idx3, table)


def kernel(text, text_lengths, embedding_weight):
  del text_lengths
  b, s = text.shape
  total = b * s
  assert total % (_NW * _CHUNK * _NBUF) == 0
  n_chunks = total // (_NW * _CHUNK)
  idx3 = text.reshape(_NW, n_chunks, _CHUNK).astype(jnp.int32)
  out = _embed_lookup(idx3, embedding_weight, n_chunks)
  return out.reshape(b, s, _D)


# speedup vs baseline: 2.2951x; 1.0014x over previous
import functools

import jax
import jax.numpy as jnp
from jax import lax
from jax.experimental import pallas as pl
from jax.experimental.pallas import tpu as pltpu
from jax.experimental.pallas import tpu_sc as plsc

_D = 64
_PAD = 0
_NC = 2
_NS = 16
_NW = _NC * _NS
_CHUNK = 200
_NBUF = 4
_LOOK = 2
_GOFF = tuple(range(0, _CHUNK - 15, 16)) + ((_CHUNK - 16,)
                                            if _CHUNK % 16 else ())


def _embed_lookup(idx3, table, n_chunks, bsz, seq):
  mesh = plsc.VectorSubcoreMesh(core_axis_name="c", subcore_axis_name="s")

  @functools.partial(
      pl.kernel,
      out_type=jax.ShapeDtypeStruct((bsz, seq, _D), jnp.float32),
      mesh=mesh,
      compiler_params=pltpu.CompilerParams(
          needs_layout_passes=False, use_tc_tiling_on_sc=False),
      scratch_types=[
          pltpu.VMEM((n_chunks, _CHUNK), jnp.int32),
          [pltpu.VMEM((_CHUNK, _D), jnp.float32) for _ in range(_NBUF)],
          pltpu.VMEM((16,), jnp.int32),
          [pltpu.SemaphoreType.DMA for _ in range(_NBUF)],
          [pltpu.SemaphoreType.DMA for _ in range(_NBUF)],
      ],
  )
  def run(idx_hbm, table_hbm, out_hbm, idx_v, rows, flag_v, gsems, osems):
    wid = lax.axis_index("s") * _NC + lax.axis_index("c")
    pltpu.sync_copy(idx_hbm.at[wid], idx_v)

    def fire(j, s):
      pltpu.make_async_copy(table_hbm.at[idx_v.at[j]], rows[s], gsems[s]).start()

    def out_start(j, s):
      pltpu.make_async_copy(rows[s], out_hbm.at[wid * n_chunks + j],
                            osems[s]).start()

    def out_wait(j, s):
      pltpu.make_async_copy(rows[s], out_hbm.at[wid * n_chunks + j],
                            osems[s]).wait()

    def handle(j, s):
      pltpu.make_async_copy(table_hbm.at[idx_v.at[j]], rows[s], gsems[s]).wait()
      idx_row = idx_v.at[j]
      msk_acc = idx_row[pl.ds(_GOFF[0], 16)] == _PAD
      for off in _GOFF[1:]:
        msk_acc = msk_acc | (idx_row[pl.ds(off, 16)] == _PAD)
      flag_v[...] = jnp.zeros((16,), jnp.int32)
      plsc.store_scatter(flag_v.at[...], [jnp.zeros((16,), jnp.int32)],
                         jnp.ones((16,), jnp.int32), mask=msk_acc)
      nz = flag_v[...][0]

      @pl.when(nz != 0)
      def _fixup():
        zero16 = jnp.zeros((16,), jnp.float32)
        for off in _GOFF:
          v = idx_row[pl.ds(off, 16)]
          msk = v == _PAD
          rowv = off + lax.iota(jnp.int32, 16)

          def cbody(c, carry, _rowv=rowv, _msk=msk):
            colv = jnp.zeros((16,), jnp.int32) + c
            plsc.store_scatter(rows[s].at[...], [_rowv, colv], zero16,
                               mask=_msk)
            return carry

          lax.fori_loop(0, _D, cbody, 0)

      out_start(j, s)

    for j in range(_LOOK):
      fire(j, j % _NBUF)

    def body4(t, carry):
      for b in range(_NBUF):
        j = _NBUF * t + b
        handle(j, b)
        g = j + _LOOK
        s2 = (b + _LOOK) % _NBUF

        @pl.when(g < n_chunks)
        def _next(_g=g, _s2=s2):
          @pl.when(_g >= _NBUF)
          def _drain():
            out_wait(_g - _NBUF, _s2)

          fire(_g, _s2)

      return carry

    lax.fori_loop(0, n_chunks // _NBUF, body4, 0)

    for b in range(_NBUF):
      out_wait(n_chunks - _NBUF + b, b)

  return run(idx3, table)


def kernel(text, text_lengths, embedding_weight):
  del text_lengths
  b, s = text.shape
  assert s == _CHUNK and b % (_NW * _NBUF) == 0
  n_chunks = b // _NW
  idx3 = text.reshape(_NW, n_chunks, _CHUNK).astype(jnp.int32)
  return _embed_lookup(idx3, embedding_weight, n_chunks, b, s)
